# baseline (speedup 1.0000x reference)
import functools

import jax
import jax.numpy as jnp
from jax import lax
from jax.experimental import pallas as pl
from jax.experimental.pallas import tpu as pltpu
from jax.experimental.pallas import tpu_sc as plsc

NC = 2
NS = 16
L = 16
NW = NC * NS

B = 16384
D = 64
BPW = B // NW
GROUPS = BPW // L

_mesh = plsc.VectorSubcoreMesh(core_axis_name="c", subcore_axis_name="s")


@functools.partial(
    pl.kernel,
    out_type=jax.ShapeDtypeStruct((B,), jnp.float32),
    mesh=_mesh,
    compiler_params=pltpu.CompilerParams(
        needs_layout_passes=False, use_tc_tiling_on_sc=False),
    scratch_types=[
        pltpu.VMEM((BPW,), jnp.int32),
        pltpu.VMEM((BPW,), jnp.int32),
        pltpu.VMEM((BPW, D), jnp.float32),
        pltpu.VMEM((BPW, D), jnp.float32),
        pltpu.VMEM((BPW,), jnp.float32),
        pltpu.VMEM((BPW,), jnp.float32),
        pltpu.VMEM((L,), jnp.float32),
        pltpu.VMEM((BPW,), jnp.float32),
        pltpu.SemaphoreType.DMA,
        pltpu.SemaphoreType.DMA,
        pltpu.SemaphoreType.DMA,
        pltpu.SemaphoreType.DMA,
    ],
)
def _rec_kernel(users, movies, U, M, bu, bm, b0v, out,
                uidx, midx, urows, mrows, burow, bmrow, b0_v, outbuf,
                sem_u, sem_m, sem_bu, sem_bm):
    wid = lax.axis_index("s") * NC + lax.axis_index("c")
    base = wid * BPW
    pltpu.sync_copy(users.at[pl.ds(base, BPW)], uidx)
    pltpu.sync_copy(movies.at[pl.ds(base, BPW)], midx)
    cu = pltpu.async_copy(U.at[uidx], urows, sem_u)
    cm = pltpu.async_copy(M.at[midx], mrows, sem_m)
    cbu = pltpu.async_copy(bu.at[uidx], burow, sem_bu)
    cbm = pltpu.async_copy(bm.at[midx], bmrow, sem_bm)
    pltpu.sync_copy(b0v, b0_v)
    cu.wait()
    cm.wait()
    cbu.wait()
    cbm.wait()
    b0x = b0_v[...]

    def group(g, carry):
        rows = g * L + lax.iota(jnp.int32, L)
        acc = jnp.zeros((L,), jnp.float32)
        for j in range(D):
            cols = jnp.full((L,), j, jnp.int32)
            uv = plsc.load_gather(urows, [rows, cols])
            mv = plsc.load_gather(mrows, [rows, cols])
            acc = acc + uv * mv
        r = acc + burow[pl.ds(g * L, L)] + bmrow[pl.ds(g * L, L)] + b0x
        outbuf[pl.ds(g * L, L)] = 5.0 / (1.0 + jnp.exp(-r))
        return carry

    lax.fori_loop(0, GROUPS, group, 0)
    pltpu.sync_copy(outbuf, out.at[pl.ds(base, BPW)])


def kernel(users, movies, U, M, bu, bm, b0):
    b0v = jnp.full((L,), b0, jnp.float32)
    return _rec_kernel(users.astype(jnp.int32), movies.astype(jnp.int32),
                       U, M, bu, bm, b0v)
